# baseline (speedup 1.0000x reference)
import functools

import jax
import jax.numpy as jnp
from jax import lax
from jax.experimental import pallas as pl
from jax.experimental.pallas import tpu as pltpu
from jax.experimental.pallas import tpu_sc as plsc

_G = 3
_NC, _NS = 2, 16
_NW = _NC * _NS
_C = 10240


def _make_sc_mean(n_total, off, n_words):
    pw = n_words // _NW
    nch = pw // _C
    mesh = plsc.VectorSubcoreMesh(core_axis_name="c", subcore_axis_name="s")

    @functools.partial(
        pl.kernel,
        mesh=mesh,
        out_type=jax.ShapeDtypeStruct((n_words,), jnp.float32),
        scratch_types=(
            [pltpu.VMEM((_C,), jnp.float32) for _ in range(2 * _G + 2)]
            + [pltpu.SemaphoreType.DMA] * 4
        ),
    )
    def sc_mean(x_hbm, o_hbm, a0, b0, c0_, a1, b1, c1_, ov0, ov1, s0, s1, t0, t1):
        wid = lax.axis_index("s") * _NC + lax.axis_index("c")
        base0 = wid * pw
        in_slots = ((a0, b0, c0_), (a1, b1, c1_))
        out_slots = (ov0, ov1)
        out_sems = (t0, t1)

        def issue_in(slot, sem, base):
            for g in range(_G):
                pltpu.async_copy(
                    x_hbm.at[pl.ds(g * n_total + off + base, _C)],
                    in_slots[slot][g],
                    sem,
                )

        def drain_in(slot, sem):
            for g in range(_G):
                pltpu.make_async_copy(
                    x_hbm.at[pl.ds(0, _C)], in_slots[slot][g], sem
                ).wait()

        def compute(slot):
            bufs = in_slots[slot]
            ov = out_slots[slot]

            @plsc.parallel_loop(0, _C, step=16, unroll=8)
            def _body(i):
                sl = pl.ds(i, 16)
                ov[sl] = (bufs[0][sl] + bufs[1][sl] + bufs[2][sl]) * (1.0 / _G)

        def issue_out(slot, base):
            pltpu.async_copy(out_slots[slot], o_hbm.at[pl.ds(base, _C)], out_sems[slot])

        def drain_out(slot):
            pltpu.make_async_copy(
                out_slots[slot], o_hbm.at[pl.ds(0, _C)], out_sems[slot]
            ).wait()

        issue_in(0, s0, base0)

        def pair(ii, _):
            c0 = base0 + (2 * ii) * _C
            issue_in(1, s1, c0 + _C)
            drain_in(0, s0)

            @pl.when(ii > 0)
            def _w0():
                drain_out(0)

            compute(0)
            issue_out(0, c0)

            @pl.when(2 * ii + 2 < nch)
            def _prefetch():
                issue_in(0, s0, c0 + 2 * _C)

            drain_in(1, s1)

            @pl.when(ii > 0)
            def _w1():
                drain_out(1)

            compute(1)
            issue_out(1, c0 + _C)
            return 0

        lax.fori_loop(0, nch // 2, pair, 0)
        drain_out(0)
        drain_out(1)

    return sc_mean


def _tc_mean(x, rows_out, block_rows):
    g, rows, d = x.shape

    def body(x_ref, o_ref):
        o_ref[...] = (x_ref[0] + x_ref[1] + x_ref[2]) * (1.0 / _G)

    return pl.pallas_call(
        body,
        grid=(rows_out // block_rows,),
        in_specs=[pl.BlockSpec((g, block_rows, d), lambda i: (0, i, 0))],
        out_specs=pl.BlockSpec((block_rows, d), lambda i: (i, 0)),
        out_shape=jax.ShapeDtypeStruct((rows_out, d), jnp.float32),
    )(x)


def kernel(time_emb, expert_embeddings, time_step, total_steps):
    del time_emb, time_step, total_steps
    G, B, L, D = expert_embeddings.shape
    rows = B * L
    rows_tc = 7168
    n_total = rows * D
    off = rows_tc * D
    x = expert_embeddings.reshape(G, rows, D)
    x_flat = expert_embeddings.reshape(G * n_total)
    tc_out = _tc_mean(x, rows_tc, 512)
    sc_out = _make_sc_mean(n_total, off, n_total - off)(x_flat)
    out = jnp.concatenate([tc_out, sc_out.reshape(rows - rows_tc, D)], axis=0)
    return out.reshape(B, L, D)


# speedup vs baseline: 4.1693x; 4.1693x over previous
import functools

import jax
import jax.numpy as jnp
from jax import lax
from jax.experimental import pallas as pl
from jax.experimental.pallas import tpu as pltpu
from jax.experimental.pallas import tpu_sc as plsc

_G = 3
_NC, _NS = 2, 16
_NW = _NC * _NS
_C = 10240


def _make_sc_mean(n_total, off, n_words):
    pw = n_words // _NW
    nch = pw // _C
    mesh = plsc.VectorSubcoreMesh(core_axis_name="c", subcore_axis_name="s")

    @functools.partial(
        pl.kernel,
        mesh=mesh,
        out_type=jax.ShapeDtypeStruct((n_words,), jnp.float32),
        scratch_types=(
            [pltpu.VMEM((_C,), jnp.float32) for _ in range(2 * _G + 2)]
            + [pltpu.SemaphoreType.DMA] * 4
        ),
    )
    def sc_mean(x_hbm, o_hbm, a0, b0, c0_, a1, b1, c1_, ov0, ov1, s0, s1, t0, t1):
        wid = lax.axis_index("s") * _NC + lax.axis_index("c")
        base0 = wid * pw
        in_slots = ((a0, b0, c0_), (a1, b1, c1_))
        out_slots = (ov0, ov1)
        out_sems = (t0, t1)

        def issue_in(slot, sem, base):
            for g in range(_G):
                pltpu.async_copy(
                    x_hbm.at[pl.ds(g * n_total + off + base, _C)],
                    in_slots[slot][g],
                    sem,
                )

        def drain_in(slot, sem):
            for g in range(_G):
                pltpu.make_async_copy(
                    x_hbm.at[pl.ds(0, _C)], in_slots[slot][g], sem
                ).wait()

        def compute(slot):
            bufs = in_slots[slot]
            ov = out_slots[slot]

            @plsc.parallel_loop(0, _C, step=16, unroll=8)
            def _body(i):
                sl = pl.ds(i, 16)
                ov[sl] = (bufs[0][sl] + bufs[1][sl] + bufs[2][sl]) * (1.0 / _G)

        def issue_out(slot, base):
            pltpu.async_copy(out_slots[slot], o_hbm.at[pl.ds(base, _C)], out_sems[slot])

        def drain_out(slot):
            pltpu.make_async_copy(
                out_slots[slot], o_hbm.at[pl.ds(0, _C)], out_sems[slot]
            ).wait()

        issue_in(0, s0, base0)

        def pair(ii, _):
            c0 = base0 + (2 * ii) * _C
            issue_in(1, s1, c0 + _C)
            drain_in(0, s0)

            @pl.when(ii > 0)
            def _w0():
                drain_out(0)

            compute(0)
            issue_out(0, c0)

            @pl.when(2 * ii + 2 < nch)
            def _prefetch():
                issue_in(0, s0, c0 + 2 * _C)

            drain_in(1, s1)

            @pl.when(ii > 0)
            def _w1():
                drain_out(1)

            compute(1)
            issue_out(1, c0 + _C)
            return 0

        lax.fori_loop(0, nch // 2, pair, 0)
        drain_out(0)
        drain_out(1)

    return sc_mean


def _tc_mean(x, rows_out, block_rows):
    g, rows, d = x.shape

    def body(x_ref, o_ref):
        o_ref[...] = (x_ref[0] + x_ref[1] + x_ref[2]) * (1.0 / _G)

    return pl.pallas_call(
        body,
        grid=(rows_out // block_rows,),
        in_specs=[pl.BlockSpec((g, block_rows, d), lambda i: (0, i, 0))],
        out_specs=pl.BlockSpec((block_rows, d), lambda i: (i, 0)),
        out_shape=jax.ShapeDtypeStruct((rows_out, d), jnp.float32),
        compiler_params=pltpu.CompilerParams(
            dimension_semantics=("parallel",),
        ),
    )(x)


def kernel(time_emb, expert_embeddings, time_step, total_steps):
    del time_emb, time_step, total_steps
    G, B, L, D = expert_embeddings.shape
    rows = B * L
    x = expert_embeddings.reshape(G, rows, D)
    out = _tc_mean(x, rows, 512)
    return out.reshape(B, L, D)
